# Initial kernel scaffold; baseline (speedup 1.0000x reference)
import jax
import jax.numpy as jnp
from jax.experimental import pallas as pl


def kernel(item_seq, item_seq_len):
    raise NotImplementedError("write your pallas kernel here")


# speedup vs baseline: 1.9582x; 1.9582x over previous
import functools

import jax
import jax.numpy as jnp
from jax import lax
from jax.experimental import pallas as pl

_REORDER_RATIO = 0.5
_MIN_W = 2
_MAX_W = 5


def _row_uniforms(key, batch):
    base = key
    keys = jax.vmap(lambda i: jax.random.fold_in(base, i))(
        jnp.arange(batch, dtype=jnp.int32))

    def draws(k):
        ku, kw, ks, kp = jax.random.split(k, 4)
        return (jax.random.uniform(ku), jax.random.uniform(kw),
                jax.random.uniform(ks), jax.random.uniform(kp, (_MAX_W,)))

    uu, uw, us, r = jax.vmap(draws)(keys)
    return uu, uw, us, r


def _reorder_block(seq_ref, aux_ref, out_ref, *, rows, length):
    seq = seq_ref[...]
    aux = aux_ref[...]
    slen = aux[:, 0:1]
    uu = aux[:, 1:2]
    uw = aux[:, 2:3]
    us = aux[:, 3:4]

    pos = lax.broadcasted_iota(jnp.int32, (rows, length), 1)
    valid = (pos < slen.astype(jnp.int32)) & (seq != 0)
    vf = valid.astype(jnp.float32)

    qi = lax.broadcasted_iota(jnp.int32, (length, length), 0)
    pi = lax.broadcasted_iota(jnp.int32, (length, length), 1)
    tri = (qi <= pi).astype(jnp.float32)
    csum = lax.dot_general(vf, tri, (((1,), (0,)), ((), ())),
                           preferred_element_type=jnp.float32)

    nv = csum[:, length - 1:length]

    apply_aug = (slen > _MIN_W) & (uu <= _REORDER_RATIO) & (nv >= _MIN_W)
    maxp = jnp.minimum(nv, float(_MAX_W))
    span = jnp.maximum(maxp - (_MIN_W - 1), 1.0)
    ws = _MIN_W + jnp.floor(uw * span)
    ws = jnp.clip(ws, float(_MIN_W), jnp.maximum(maxp, float(_MIN_W)))
    max_start = jnp.maximum(nv - ws + 1.0, 1.0)
    start = jnp.floor(us * max_start)

    masks = []
    witems = []
    for w in range(_MAX_W):
        tgt = jnp.clip(start + float(w), 0.0, float(length - 1))
        m = valid & (csum == tgt + 1.0)
        masks.append(m)
        witems.append(jnp.sum(jnp.where(m, seq, 0), axis=1, keepdims=True))

    inf = jnp.float32(jnp.inf)
    rs = [jnp.where(float(w) < ws, aux[:, 4 + w:5 + w], inf)
          for w in range(_MAX_W)]
    ranks = []
    for i in range(_MAX_W):
        rk = jnp.zeros_like(rs[i])
        for k in range(_MAX_W):
            lt = rs[k] < rs[i]
            tie = (rs[k] == rs[i]) & (k < i)
            rk = rk + jnp.where(lt | tie, 1.0, 0.0)
        ranks.append(rk)

    shuf = []
    for j in range(_MAX_W):
        s = jnp.zeros_like(witems[0])
        for i in range(_MAX_W):
            s = s + jnp.where(ranks[i] == float(j), witems[i], 0)
        shuf.append(s)

    out = seq
    for w in range(_MAX_W):
        do_w = apply_aug & (float(w) < ws)
        out = jnp.where(do_w & masks[w], shuf[w], out)
    out_ref[...] = out


def _run(item_seq, aux, *, rows, interpret=False):
    batch, length = item_seq.shape
    grid = batch // rows
    f = pl.pallas_call(
        functools.partial(_reorder_block, rows=rows, length=length),
        grid=(grid,),
        in_specs=[
            pl.BlockSpec((rows, length), lambda i: (i, 0)),
            pl.BlockSpec((rows, 16), lambda i: (i, 0)),
        ],
        out_specs=pl.BlockSpec((rows, length), lambda i: (i, 0)),
        out_shape=jax.ShapeDtypeStruct((batch, length), jnp.int32),
        interpret=interpret,
    )
    return f(item_seq, aux)


def kernel(item_seq, item_seq_len):
    batch, _ = item_seq.shape
    uu, uw, us, r = _row_uniforms(jax.random.key(1), batch)
    aux = jnp.zeros((batch, 16), jnp.float32)
    aux = aux.at[:, 0].set(item_seq_len.astype(jnp.float32))
    aux = aux.at[:, 1].set(uu)
    aux = aux.at[:, 2].set(uw)
    aux = aux.at[:, 3].set(us)
    aux = aux.at[:, 4:4 + _MAX_W].set(r)
    out = _run(item_seq, aux, rows=256)
    return out, item_seq_len


# speedup vs baseline: 2.0540x; 1.0489x over previous
import functools

import jax
import jax.numpy as jnp
from jax import lax
from jax.experimental import pallas as pl

_REORDER_RATIO = 0.5
_MIN_W = 2
_MAX_W = 5


def _row_uniforms(key, batch):
    base = key
    keys = jax.vmap(lambda i: jax.random.fold_in(base, i))(
        jnp.arange(batch, dtype=jnp.int32))

    def draws(k):
        ku, kw, ks, kp = jax.random.split(k, 4)
        return (jax.random.uniform(ku), jax.random.uniform(kw),
                jax.random.uniform(ks), jax.random.uniform(kp, (_MAX_W,)))

    uu, uw, us, r = jax.vmap(draws)(keys)
    return uu, uw, us, r


def _reorder_block(seq_ref, aux_ref, out_ref, *, rows, length):
    seq = seq_ref[...]
    aux = aux_ref[...]
    slen = aux[:, 0:1]
    uu = aux[:, 1:2]
    uw = aux[:, 2:3]
    us = aux[:, 3:4]

    pos = lax.broadcasted_iota(jnp.int32, (rows, length), 1)
    valid = (pos < slen.astype(jnp.int32)) & (seq != 0)
    vf = valid.astype(jnp.float32)

    qi = lax.broadcasted_iota(jnp.int32, (length, length), 0)
    pi = lax.broadcasted_iota(jnp.int32, (length, length), 1)
    tri = (qi <= pi).astype(jnp.bfloat16)
    csum = lax.dot_general(vf.astype(jnp.bfloat16), tri,
                           (((1,), (0,)), ((), ())),
                           preferred_element_type=jnp.float32)

    nv = csum[:, length - 1:length]

    apply_aug = (slen > _MIN_W) & (uu <= _REORDER_RATIO) & (nv >= _MIN_W)
    maxp = jnp.minimum(nv, float(_MAX_W))
    span = jnp.maximum(maxp - (_MIN_W - 1), 1.0)
    ws = _MIN_W + jnp.floor(uw * span)
    ws = jnp.clip(ws, float(_MIN_W), jnp.maximum(maxp, float(_MIN_W)))
    max_start = jnp.maximum(nv - ws + 1.0, 1.0)
    start = jnp.floor(us * max_start)

    masks = []
    witems = []
    for w in range(_MAX_W):
        tgt = jnp.clip(start + float(w), 0.0, float(length - 1))
        m = valid & (csum == tgt + 1.0)
        masks.append(m)
        witems.append(jnp.sum(jnp.where(m, seq, 0), axis=1, keepdims=True))

    inf = jnp.float32(jnp.inf)
    rs = [jnp.where(float(w) < ws, aux[:, 4 + w:5 + w], inf)
          for w in range(_MAX_W)]
    ranks = []
    for i in range(_MAX_W):
        rk = jnp.zeros_like(rs[i])
        for k in range(_MAX_W):
            lt = rs[k] < rs[i]
            tie = (rs[k] == rs[i]) & (k < i)
            rk = rk + jnp.where(lt | tie, 1.0, 0.0)
        ranks.append(rk)

    shuf = []
    for j in range(_MAX_W):
        s = jnp.zeros_like(witems[0])
        for i in range(_MAX_W):
            s = s + jnp.where(ranks[i] == float(j), witems[i], 0)
        shuf.append(s)

    out = seq
    for w in range(_MAX_W):
        do_w = apply_aug & (float(w) < ws)
        out = jnp.where(do_w & masks[w], shuf[w], out)
    out_ref[...] = out


def _run(item_seq, aux, *, rows, interpret=False):
    batch, length = item_seq.shape
    grid = batch // rows
    f = pl.pallas_call(
        functools.partial(_reorder_block, rows=rows, length=length),
        grid=(grid,),
        in_specs=[
            pl.BlockSpec((rows, length), lambda i: (i, 0)),
            pl.BlockSpec((rows, 16), lambda i: (i, 0)),
        ],
        out_specs=pl.BlockSpec((rows, length), lambda i: (i, 0)),
        out_shape=jax.ShapeDtypeStruct((batch, length), jnp.int32),
        interpret=interpret,
    )
    return f(item_seq, aux)


def kernel(item_seq, item_seq_len):
    batch, _ = item_seq.shape
    uu, uw, us, r = _row_uniforms(jax.random.key(1), batch)
    aux = jnp.zeros((batch, 16), jnp.float32)
    aux = aux.at[:, 0].set(item_seq_len.astype(jnp.float32))
    aux = aux.at[:, 1].set(uu)
    aux = aux.at[:, 2].set(uw)
    aux = aux.at[:, 3].set(us)
    aux = aux.at[:, 4:4 + _MAX_W].set(r)
    out = _run(item_seq, aux, rows=512)
    return out, item_seq_len


# speedup vs baseline: 2.4028x; 1.1698x over previous
import functools

import jax
import jax.numpy as jnp
from jax import lax
from jax.experimental import pallas as pl
from jax.experimental.pallas import tpu as pltpu
from jax.experimental.pallas import tpu_sc as plsc

_REORDER_RATIO = 0.5
_MIN_W = 2
_MAX_W = 5
_LANES = 16


def _row_uniforms(key, batch):
    base = key
    keys = jax.vmap(lambda i: jax.random.fold_in(base, i))(
        jnp.arange(batch, dtype=jnp.int32))

    def draws(k):
        ku, kw, ks, kp = jax.random.split(k, 4)
        return (jax.random.uniform(ku), jax.random.uniform(kw),
                jax.random.uniform(ks), jax.random.uniform(kp, (_MAX_W,)))

    uu, uw, us, r = jax.vmap(draws)(keys)
    return uu, uw, us, r


def _build_aux(item_seq_len, batch):
    uu, uw, us, r = _row_uniforms(jax.random.key(1), batch)
    slen = item_seq_len.astype(jnp.int32)
    flag = ((slen > _MIN_W) & (uu <= _REORDER_RATIO)).astype(jnp.int32)
    aux_f = jnp.zeros((batch, 16), jnp.float32)
    aux_f = aux_f.at[:, 0:_MAX_W].set(r)
    aux_f = aux_f.at[:, 5].set(uw)
    aux_f = aux_f.at[:, 6].set(us)
    aux_i = jnp.zeros((batch, 16), jnp.int32)
    aux_i = aux_i.at[:, 0].set(flag)
    aux_i = aux_i.at[:, 1].set(slen)
    return aux_f, aux_i


_GROUP = 128


def _sc_body(seq_hbm, auxf_hbm, auxi_hbm, out_hbm, seq_v, auxf_v, auxi_v,
             pk_v, *, n_rows, length, n_workers):
    rows_per = n_rows // n_workers
    n_groups = rows_per // _GROUP
    wid = lax.axis_index("s") * 2 + lax.axis_index("c")
    base = wid * rows_per

    lanes = lax.iota(jnp.int32, _LANES)
    inf_v = jnp.full((_LANES,), jnp.inf, jnp.float32)

    def group_fn(g, _):
        gbase = base + g * _GROUP
        pltpu.sync_copy(seq_hbm.at[pl.ds(gbase, _GROUP)], seq_v)
        pltpu.sync_copy(auxf_hbm.at[pl.ds(gbase, _GROUP)], auxf_v)
        pltpu.sync_copy(auxi_hbm.at[pl.ds(gbase, _GROUP)], auxi_v)
        _sc_group(seq_v, auxf_v, auxi_v, pk_v, lanes, inf_v, length)
        pltpu.sync_copy(seq_v, out_hbm.at[pl.ds(gbase, _GROUP)])

    lax.fori_loop(0, n_groups, group_fn, None)


def _sc_group(seq_v, auxf_v, auxi_v, pk_v, lanes, inf_v, length):
    big_v = jnp.full((_LANES,), jnp.int32(1 << 30), jnp.int32)

    def row_fn(r, _):
        auxirow = auxi_v[r, :]
        flag = auxirow[0]

        @pl.when(flag != 0)
        def _process():
            auxrow_f = auxf_v[r, :]
            slen = auxirow[1]
            slen_v = jnp.full((_LANES,), slen, jnp.int32)
            nch = (slen + (_LANES - 1)) // _LANES

            def chunk_fn(c, carry):
                off = c * _LANES
                v = seq_v[r, pl.ds(off, _LANES)]
                posv = lanes + off
                m = (v != 0) & (posv < slen_v)
                packed, _ = plsc.sort_key_val(jnp.where(m, posv, big_v),
                                              posv)
                cnt = plsc.all_reduce_population_count(m)
                plsc.store_scatter(pk_v, [carry + lanes], packed,
                                   mask=lanes < cnt)
                return carry + cnt

            nv = lax.fori_loop(0, nch, chunk_fn,
                               jnp.zeros((_LANES,), jnp.int32))

            nv_f = nv.astype(jnp.float32)
            uw_f = jnp.full((_LANES,), auxrow_f[5], jnp.float32)
            us_f = jnp.full((_LANES,), auxrow_f[6], jnp.float32)

            maxp = jnp.minimum(nv_f, float(_MAX_W))
            span = jnp.maximum(maxp - (_MIN_W - 1), 1.0)
            ws = _MIN_W + (uw_f * span).astype(jnp.int32)
            ws = jnp.clip(ws, _MIN_W,
                          jnp.maximum(maxp.astype(jnp.int32), _MIN_W))
            max_start = jnp.maximum(nv_f - ws.astype(jnp.float32) + 1.0, 1.0)
            start = (us_f * max_start).astype(jnp.int32)

            tgt = jnp.clip(start + lanes, 0, length - 1)
            win_pos = plsc.load_gather(pk_v, [tgt])
            win_pos = jnp.clip(win_pos, 0, length - 1)
            r_splat = jnp.full((_LANES,), r, jnp.int32)
            win_items = plsc.load_gather(seq_v, [r_splat, win_pos])

            in_win = lanes < ws
            key = jnp.where(in_win, auxrow_f, inf_v)
            _, shuffled = plsc.sort_key_val(key, win_items)
            do_write = in_win & (nv >= _MIN_W)
            plsc.store_scatter(seq_v, [r_splat, win_pos], shuffled,
                               mask=do_write)

    lax.fori_loop(0, _GROUP, row_fn, None)


def _run_sc(item_seq, aux_f, aux_i):
    batch, length = item_seq.shape
    n_workers = 32
    rows_per = batch // n_workers
    mesh = plsc.VectorSubcoreMesh(core_axis_name="c", subcore_axis_name="s")
    f = pl.kernel(
        functools.partial(_sc_body, n_rows=batch, length=length,
                          n_workers=n_workers),
        mesh=mesh,
        compiler_params=pltpu.CompilerParams(use_tc_tiling_on_sc=False, needs_layout_passes=False),
        out_type=jax.ShapeDtypeStruct((batch, 208), jnp.int32),
        scratch_types=[
            pltpu.VMEM((_GROUP, 208), jnp.int32),
            pltpu.VMEM((_GROUP, 16), jnp.float32),
            pltpu.VMEM((_GROUP, 16), jnp.int32),
            pltpu.VMEM((256,), jnp.int32),
        ],
    )
    seq_pad = jnp.pad(item_seq, ((0, 0), (0, 208 - length)))
    return f(seq_pad, aux_f, aux_i)[:, :length]


def kernel(item_seq, item_seq_len):
    batch, _ = item_seq.shape
    aux_f, aux_i = _build_aux(item_seq_len, batch)
    out = _run_sc(item_seq, aux_f, aux_i)
    return out, item_seq_len


# speedup vs baseline: 4.2308x; 1.7607x over previous
import functools

import jax
import jax.numpy as jnp
from jax import lax
from jax.experimental import pallas as pl
from jax.experimental.pallas import tpu as pltpu
from jax.experimental.pallas import tpu_sc as plsc

_REORDER_RATIO = 0.5
_MIN_W = 2
_MAX_W = 5
_LANES = 16


def _row_uniforms(key, batch):
    base = key
    keys = jax.vmap(lambda i: jax.random.fold_in(base, i))(
        jnp.arange(batch, dtype=jnp.int32))

    def draws(k):
        ku, kw, ks, kp = jax.random.split(k, 4)
        return (jax.random.uniform(ku), jax.random.uniform(kw),
                jax.random.uniform(ks), jax.random.uniform(kp, (_MAX_W,)))

    uu, uw, us, r = jax.vmap(draws)(keys)
    return uu, uw, us, r


def _build_aux(item_seq_len, batch):
    uu, uw, us, r = _row_uniforms(jax.random.key(1), batch)
    slen = item_seq_len.astype(jnp.int32)
    flag = ((slen > _MIN_W) & (uu <= _REORDER_RATIO)).astype(jnp.int32)
    zf = jnp.zeros((batch, 9), jnp.float32)
    aux_f = jnp.concatenate([r, uw[:, None], us[:, None], zf], axis=1)
    zi = jnp.zeros((batch, 14), jnp.int32)
    aux_i = jnp.concatenate([flag[:, None], slen[:, None], zi], axis=1)
    return aux_f, aux_i


_GROUP = 128


def _sc_body(seq_hbm, auxf_hbm, auxi_hbm, out_hbm, seq_v, auxf_v, auxi_v,
             pk_v, *, n_rows, length, n_workers):
    rows_per = n_rows // n_workers
    n_groups = rows_per // _GROUP
    wid = lax.axis_index("s") * 2 + lax.axis_index("c")
    base = wid * rows_per

    lanes = lax.iota(jnp.int32, _LANES)
    inf_v = jnp.full((_LANES,), jnp.inf, jnp.float32)

    def group_fn(g, _):
        gbase = base + g * _GROUP
        pltpu.sync_copy(seq_hbm.at[pl.ds(gbase, _GROUP)], seq_v)
        pltpu.sync_copy(auxf_hbm.at[pl.ds(gbase, _GROUP)], auxf_v)
        pltpu.sync_copy(auxi_hbm.at[pl.ds(gbase, _GROUP)], auxi_v)
        _sc_group(seq_v, auxf_v, auxi_v, pk_v, lanes, inf_v, length)
        pltpu.sync_copy(seq_v, out_hbm.at[pl.ds(gbase, _GROUP)])

    lax.fori_loop(0, n_groups, group_fn, None)


def _sc_group(seq_v, auxf_v, auxi_v, pk_v, lanes, inf_v, length):
    big_v = jnp.full((_LANES,), jnp.int32(1 << 30), jnp.int32)

    def row_fn(r, _):
        auxirow = auxi_v[r, :]
        flag = auxirow[0]

        @pl.when(flag != 0)
        def _process():
            auxrow_f = auxf_v[r, :]
            slen = auxirow[1]
            slen_v = jnp.full((_LANES,), slen, jnp.int32)
            nch = jnp.minimum((slen + (_LANES - 1)) // _LANES, 12)

            def compact(off, posv, carry):
                v = seq_v[r, pl.ds(off, _LANES)]
                m = (v != 0) & (posv < slen_v)
                packed, _ = plsc.sort_key_val(jnp.where(m, posv, big_v),
                                              posv)
                cnt = plsc.all_reduce_population_count(m)
                plsc.store_scatter(pk_v, [carry + lanes], packed,
                                   mask=lanes < cnt)
                return carry + cnt

            def chunk_fn(c, carry):
                off = c * _LANES
                return compact(off, lanes + off, carry)

            nv0 = lax.fori_loop(0, nch, chunk_fn,
                                jnp.zeros((_LANES,), jnp.int32))

            def tail_fn(carry):
                off = length - _LANES
                posv = lanes + off
                v = seq_v[r, pl.ds(off, _LANES)]
                m = (v != 0) & (posv < slen_v) & (posv >= 12 * _LANES)
                packed, _ = plsc.sort_key_val(jnp.where(m, posv, big_v),
                                              posv)
                cnt = plsc.all_reduce_population_count(m)
                plsc.store_scatter(pk_v, [carry + lanes], packed,
                                   mask=lanes < cnt)
                return carry + cnt

            nv = lax.cond(slen > 12 * _LANES, tail_fn, lambda c: c, nv0)

            nv_f = nv.astype(jnp.float32)
            uw_f = jnp.full((_LANES,), auxrow_f[5], jnp.float32)
            us_f = jnp.full((_LANES,), auxrow_f[6], jnp.float32)

            maxp = jnp.minimum(nv_f, float(_MAX_W))
            span = jnp.maximum(maxp - (_MIN_W - 1), 1.0)
            ws = _MIN_W + (uw_f * span).astype(jnp.int32)
            ws = jnp.clip(ws, _MIN_W,
                          jnp.maximum(maxp.astype(jnp.int32), _MIN_W))
            max_start = jnp.maximum(nv_f - ws.astype(jnp.float32) + 1.0, 1.0)
            start = (us_f * max_start).astype(jnp.int32)

            tgt = jnp.clip(start + lanes, 0, length - 1)
            win_pos = plsc.load_gather(pk_v, [tgt])
            win_pos = jnp.clip(win_pos, 0, length - 1)
            r_splat = jnp.full((_LANES,), r, jnp.int32)
            win_items = plsc.load_gather(seq_v, [r_splat, win_pos])

            in_win = lanes < ws
            key = jnp.where(in_win, auxrow_f, inf_v)
            _, shuffled = plsc.sort_key_val(key, win_items)
            do_write = in_win & (nv >= _MIN_W)
            plsc.store_scatter(seq_v, [r_splat, win_pos], shuffled,
                               mask=do_write)

    lax.fori_loop(0, _GROUP, row_fn, None)


def _run_sc(item_seq, aux_f, aux_i):
    batch, length = item_seq.shape
    n_workers = 32
    rows_per = batch // n_workers
    mesh = plsc.VectorSubcoreMesh(core_axis_name="c", subcore_axis_name="s")
    f = pl.kernel(
        functools.partial(_sc_body, n_rows=batch, length=length,
                          n_workers=n_workers),
        mesh=mesh,
        compiler_params=pltpu.CompilerParams(use_tc_tiling_on_sc=False, needs_layout_passes=False),
        out_type=jax.ShapeDtypeStruct((batch, length), jnp.int32),
        scratch_types=[
            pltpu.VMEM((_GROUP, 200), jnp.int32),
            pltpu.VMEM((_GROUP, 16), jnp.float32),
            pltpu.VMEM((_GROUP, 16), jnp.int32),
            pltpu.VMEM((256,), jnp.int32),
        ],
    )
    return f(item_seq, aux_f, aux_i)


def kernel(item_seq, item_seq_len):
    batch, _ = item_seq.shape
    aux_f, aux_i = _build_aux(item_seq_len, batch)
    out = _run_sc(item_seq, aux_f, aux_i)
    return out, item_seq_len
